# baseline (speedup 1.0000x reference)
import functools

import jax
import jax.numpy as jnp
from jax import lax
from jax.experimental import pallas as pl
from jax.experimental.pallas import tpu as pltpu
from jax.experimental.pallas import tpu_sc as plsc

N = 10000
E = 320000
D = 128
DA = 144
NP = 10240
NC = 2
NS = 16
NW = NC * NS
EPW = E // NW
CH = 80
NCH = EPW // CH
RPT = NP // NS
ZR = 40


NRB = 2
NIB = 4


def _make_seg_sum(width, tc_tiling):
    def body(table, src, dst, out_sum,
             rows0, rows1, si0, si1, si2, si3, di0, di1, di2, di3, acc,
             gs0, gs1, ss0, ss1, is0, is1, is2, is3):
        c = lax.axis_index("c")
        s = lax.axis_index("s")
        wid = c * NS + s
        rows = (rows0, rows1)
        sidx = (si0, si1, si2, si3)
        didx = (di0, di1, di2, di3)
        gsem = (gs0, gs1)
        ssem = (ss0, ss1)
        isem = (is0, is1, is2, is3)

        zrow = jnp.zeros((16,), jnp.float32)
        for r in range(CH):
            for j in range(width // 16):
                rows0[r, pl.ds(j * 16, 16)] = zrow
        base_r = s * RPT
        for k in range(RPT // CH):
            pltpu.sync_copy(rows0, acc.at[pl.ds(base_r + k * CH, CH)])
        plsc.subcore_barrier()

        def idx_start(ci, b4):
            base = wid * EPW + ci * CH
            pltpu.async_copy(src.at[pl.ds(base, CH)], sidx[b4], isem[b4])
            pltpu.async_copy(dst.at[pl.ds(base, CH)], didx[b4], isem[b4])

        def idx_wait(ci, b4):
            base = wid * EPW + ci * CH
            pltpu.make_async_copy(src.at[pl.ds(base, CH)], sidx[b4],
                                  isem[b4]).wait()
            pltpu.make_async_copy(dst.at[pl.ds(base, CH)], didx[b4],
                                  isem[b4]).wait()

        def gather_start(b4, b2):
            pltpu.async_copy(table.at[sidx[b4]], rows[b2], gsem[b2])

        def gather_wait(b4, b2):
            pltpu.make_async_copy(table.at[sidx[b4]], rows[b2],
                                  gsem[b2]).wait()

        def scatter_start(b4, b2):
            pltpu.async_copy(rows[b2], acc.at[didx[b4]], ssem[b2], add=True)

        def scatter_wait(b4, b2):
            pltpu.make_async_copy(rows[b2], acc.at[didx[b4]],
                                  ssem[b2]).wait()

        def step(ci, m4):
            m2, n2, n4, p4 = m4 % 2, (m4 + 1) % 2, (m4 + 1) % 4, (m4 + 3) % 4
            if isinstance(ci, int):
                first, n_idx, n_g = ci == 0, ci + 3 < NCH, ci + 1 < NCH
            else:
                first, n_idx, n_g = False, True, True
            if n_g:
                idx_wait(ci + 1, n4)
            if not first:
                scatter_wait(p4, n2)
            if n_idx:
                idx_start(ci + 3, p4)
            if n_g:
                gather_start(n4, n2)
            gather_wait(m4, m2)
            scatter_start(m4, m2)

        idx_start(0, 0)
        idx_start(1, 1)
        idx_start(2, 2)
        idx_wait(0, 0)
        gather_start(0, 0)
        for ci in range(4):
            step(ci, ci % 4)

        def group(g, carry):
            for jj in range(4):
                step(4 * g + jj, jj)
            return carry

        lax.fori_loop(1, 29, group, 0)

        for ci in range(116, NCH):
            step(ci, ci % 4)
        scatter_wait((NCH - 1) % 4, (NCH - 1) % 2)

        plsc.subcore_barrier()
        pltpu.sync_copy(acc.at[pl.ds(base_r, RPT)],
                        out_sum.at[c, pl.ds(base_r, RPT)])

    mesh = plsc.VectorSubcoreMesh(core_axis_name="c", subcore_axis_name="s",
                                  num_cores=NC, num_subcores=NS)
    rowbuf = pltpu.VMEM((CH, width), jnp.float32)
    ibuf = pltpu.VMEM((CH,), jnp.int32)
    dma = pltpu.SemaphoreType.DMA
    return pl.kernel(
        body,
        out_type=[jax.ShapeDtypeStruct((NC, NP, width), jnp.float32)],
        mesh=mesh,
        compiler_params=pltpu.CompilerParams(use_tc_tiling_on_sc=tc_tiling),
        scratch_types=[
            rowbuf, rowbuf,
            ibuf, ibuf, ibuf, ibuf,
            ibuf, ibuf, ibuf, ibuf,
            pltpu.VMEM_SHARED((NP, width), jnp.float32),
            dma, dma, dma, dma, dma, dma, dma, dma,
        ],
        name=f"sage_segment_sum_{width}",
    )


def _layer1_body(s_ref, xa_ref, wl_ref, wr_ref, b_ref, h_ref):
    deg = s_ref[0, :, D:D + 1] + s_ref[1, :, D:D + 1]
    deg = jnp.maximum(deg, 1.0)
    ssum = s_ref[0, :, :D] + s_ref[1, :, :D]
    agg = ssum * (1.0 / deg)
    x = xa_ref[:, :D]
    h = (jnp.dot(agg, wl_ref[...], preferred_element_type=jnp.float32)
         + jnp.dot(x, wr_ref[...], preferred_element_type=jnp.float32)
         + b_ref[...])
    h_ref[...] = jnp.maximum(h, 0.0)


def _layer2_body(s_ref, sd_ref, h_ref, wl_ref, wr_ref, b_ref, o_ref):
    deg = sd_ref[0, :, 0:1] + sd_ref[1, :, 0:1]
    deg = jnp.maximum(deg, 1.0)
    agg = (s_ref[0] + s_ref[1]) * (1.0 / deg)
    o_ref[...] = (jnp.dot(agg, wl_ref[...], preferred_element_type=jnp.float32)
                  + jnp.dot(h_ref[...], wr_ref[...],
                            preferred_element_type=jnp.float32)
                  + b_ref[...])


_RB = 512
_W_SPEC = pl.BlockSpec((D, D), lambda i: (0, 0))
_B_SPEC = pl.BlockSpec((1, D), lambda i: (0, 0))
_ROW_SPEC = pl.BlockSpec((_RB, D), lambda i: (i, 0))


@jax.jit
def _layer1(S, x_aug, W_l, W_r, b):
    return pl.pallas_call(
        _layer1_body,
        grid=(NP // _RB,),
        in_specs=[
            pl.BlockSpec((NC, _RB, DA), lambda i: (0, i, 0)),
            pl.BlockSpec((_RB, DA), lambda i: (i, 0)),
            _W_SPEC, _W_SPEC, _B_SPEC,
        ],
        out_specs=_ROW_SPEC,
        out_shape=jax.ShapeDtypeStruct((NP, D), jnp.float32),
        name="sage_dense1",
    )(S, x_aug, W_l, W_r, b)


@jax.jit
def _layer2(S, S1d, h, W_l, W_r, b):
    return pl.pallas_call(
        _layer2_body,
        grid=(NP // _RB,),
        in_specs=[
            pl.BlockSpec((NC, _RB, D), lambda i: (0, i, 0)),
            pl.BlockSpec((NC, _RB, 16), lambda i: (0, i, 0)),
            _ROW_SPEC, _W_SPEC, _W_SPEC, _B_SPEC,
        ],
        out_specs=_ROW_SPEC,
        out_shape=jax.ShapeDtypeStruct((NP, D), jnp.float32),
        name="sage_dense2",
    )(S, S1d, h, W_l, W_r, b)


@functools.cache
def _seg_sum(width, tc_tiling):
    return jax.jit(_make_seg_sum(width, tc_tiling))


def kernel(x, edge_index, W1_l, W1_r, b1, W2_l, W2_r, b2):
    src = edge_index[0].astype(jnp.int32)
    dst = edge_index[1].astype(jnp.int32)
    ones = jnp.ones((N, 1), jnp.float32)
    x_aug = jnp.concatenate(
        [x, ones, jnp.zeros((N, DA - D - 1), jnp.float32)], axis=1)
    x_aug = jnp.pad(x_aug, ((0, NP - N), (0, 0)))
    (S1,) = _seg_sum(DA, False)(x_aug, src, dst)
    h = _layer1(S1, x_aug, W1_l, W1_r, b1.reshape(1, D))
    (S2,) = _seg_sum(D, True)(h, src, dst)
    out = _layer2(S2, S1[:, :, D:], h, W2_l, W2_r, b2.reshape(1, D))
    return out[:N]


# speedup vs baseline: 1.0606x; 1.0606x over previous
import functools

import jax
import jax.numpy as jnp
from jax import lax
from jax.experimental import pallas as pl
from jax.experimental.pallas import tpu as pltpu
from jax.experimental.pallas import tpu_sc as plsc

N = 10000
E = 320000
D = 128
DA = 160
NP = 10240
NC = 2
NS = 16
NW = NC * NS
EPW = E // NW
CH = 80
NCH = EPW // CH
RPT = NP // NS


def _make_seg_sum(width):
    def body(table, src, dst, out_sum,
             rows0, rows1, si0, si1, si2, si3, di0, di1, di2, di3, acc,
             gs0, gs1, ss0, ss1, is0, is1, is2, is3):
        c = lax.axis_index("c")
        s = lax.axis_index("s")
        wid = c * NS + s
        rows = (rows0, rows1)
        sidx = (si0, si1, si2, si3)
        didx = (di0, di1, di2, di3)
        gsem = (gs0, gs1)
        ssem = (ss0, ss1)
        isem = (is0, is1, is2, is3)

        zrow = jnp.zeros((32,), jnp.bfloat16)
        for r in range(CH):
            for j in range(width // 32):
                rows0[r, pl.ds(j * 32, 32)] = zrow
        base_r = s * RPT
        for k in range(RPT // CH):
            pltpu.sync_copy(rows0, acc.at[pl.ds(base_r + k * CH, CH)])
        plsc.subcore_barrier()

        def idx_start(ci, b4):
            base = wid * EPW + ci * CH
            pltpu.async_copy(src.at[pl.ds(base, CH)], sidx[b4], isem[b4])
            pltpu.async_copy(dst.at[pl.ds(base, CH)], didx[b4], isem[b4])

        def idx_wait(ci, b4):
            base = wid * EPW + ci * CH
            pltpu.make_async_copy(src.at[pl.ds(base, CH)], sidx[b4],
                                  isem[b4]).wait()
            pltpu.make_async_copy(dst.at[pl.ds(base, CH)], didx[b4],
                                  isem[b4]).wait()

        def gather_start(b4, b2):
            pltpu.async_copy(table.at[sidx[b4]], rows[b2], gsem[b2])

        def gather_wait(b4, b2):
            pltpu.make_async_copy(table.at[sidx[b4]], rows[b2],
                                  gsem[b2]).wait()

        def scatter_start(b4, b2):
            pltpu.async_copy(rows[b2], acc.at[didx[b4]], ssem[b2], add=True)

        def scatter_wait(b4, b2):
            pltpu.make_async_copy(rows[b2], acc.at[didx[b4]],
                                  ssem[b2]).wait()

        def step(ci, m4):
            m2, n2, n4, p4 = m4 % 2, (m4 + 1) % 2, (m4 + 1) % 4, (m4 + 3) % 4
            if isinstance(ci, int):
                first, n_idx, n_g = ci == 0, ci + 3 < NCH, ci + 1 < NCH
            else:
                first, n_idx, n_g = False, True, True
            if n_g:
                idx_wait(ci + 1, n4)
            if not first:
                scatter_wait(p4, n2)
            if n_idx:
                idx_start(ci + 3, p4)
            if n_g:
                gather_start(n4, n2)
            gather_wait(m4, m2)
            scatter_start(m4, m2)

        idx_start(0, 0)
        idx_start(1, 1)
        idx_start(2, 2)
        idx_wait(0, 0)
        gather_start(0, 0)
        for ci in range(4):
            step(ci, ci % 4)

        def group(g, carry):
            for jj in range(4):
                step(4 * g + jj, jj)
            return carry

        lax.fori_loop(1, 29, group, 0)

        for ci in range(116, NCH):
            step(ci, ci % 4)
        scatter_wait((NCH - 1) % 4, (NCH - 1) % 2)

        plsc.subcore_barrier()
        pltpu.sync_copy(acc.at[pl.ds(base_r, RPT)],
                        out_sum.at[c, pl.ds(base_r, RPT)])

    mesh = plsc.VectorSubcoreMesh(core_axis_name="c", subcore_axis_name="s",
                                  num_cores=NC, num_subcores=NS)
    rowbuf = pltpu.VMEM((CH, width), jnp.bfloat16)
    ibuf = pltpu.VMEM((CH,), jnp.int32)
    dma = pltpu.SemaphoreType.DMA
    return pl.kernel(
        body,
        out_type=[jax.ShapeDtypeStruct((NC, NP, width), jnp.bfloat16)],
        mesh=mesh,
        compiler_params=pltpu.CompilerParams(use_tc_tiling_on_sc=False),
        scratch_types=[
            rowbuf, rowbuf,
            ibuf, ibuf, ibuf, ibuf,
            ibuf, ibuf, ibuf, ibuf,
            pltpu.VMEM_SHARED((NP, width), jnp.bfloat16),
            dma, dma, dma, dma, dma, dma, dma, dma,
        ],
        name=f"sage_segment_sum_{width}",
    )


def _layer1_body(s_ref, x_ref, wl_ref, wr_ref, b_ref, h_ref, h16_ref):
    s0 = s_ref[0].astype(jnp.float32)
    s1 = s_ref[1].astype(jnp.float32)
    deg = jnp.maximum(s0[:, D:D + 1] + s1[:, D:D + 1], 1.0)
    agg = (s0[:, :D] + s1[:, :D]) * (1.0 / deg)
    h = (jnp.dot(agg, wl_ref[...], preferred_element_type=jnp.float32)
         + jnp.dot(x_ref[...], wr_ref[...], preferred_element_type=jnp.float32)
         + b_ref[...])
    h = jnp.maximum(h, 0.0)
    h_ref[...] = h
    h16_ref[...] = h.astype(jnp.bfloat16)


def _layer2_body(s_ref, sd_ref, h_ref, wl_ref, wr_ref, b_ref, o_ref):
    deg = (sd_ref[0, :, 0:1] + sd_ref[1, :, 0:1]).astype(jnp.float32)
    deg = jnp.maximum(deg, 1.0)
    agg = (s_ref[0].astype(jnp.float32)
           + s_ref[1].astype(jnp.float32)) * (1.0 / deg)
    o_ref[...] = (jnp.dot(agg, wl_ref[...], preferred_element_type=jnp.float32)
                  + jnp.dot(h_ref[...], wr_ref[...],
                            preferred_element_type=jnp.float32)
                  + b_ref[...])


_RB = 512
_W_SPEC = pl.BlockSpec((D, D), lambda i: (0, 0))
_B_SPEC = pl.BlockSpec((1, D), lambda i: (0, 0))
_ROW_SPEC = pl.BlockSpec((_RB, D), lambda i: (i, 0))


@jax.jit
def _layer1(S, x_pad, W_l, W_r, b):
    return pl.pallas_call(
        _layer1_body,
        grid=(NP // _RB,),
        in_specs=[
            pl.BlockSpec((NC, _RB, DA), lambda i: (0, i, 0)),
            _ROW_SPEC, _W_SPEC, _W_SPEC, _B_SPEC,
        ],
        out_specs=[_ROW_SPEC, _ROW_SPEC],
        out_shape=[jax.ShapeDtypeStruct((NP, D), jnp.float32),
                   jax.ShapeDtypeStruct((NP, D), jnp.bfloat16)],
        name="sage_dense1",
    )(S, x_pad, W_l, W_r, b)


@jax.jit
def _layer2(S, S1d, h, W_l, W_r, b):
    return pl.pallas_call(
        _layer2_body,
        grid=(NP // _RB,),
        in_specs=[
            pl.BlockSpec((NC, _RB, D), lambda i: (0, i, 0)),
            pl.BlockSpec((NC, _RB, DA - D), lambda i: (0, i, 0)),
            _ROW_SPEC, _W_SPEC, _W_SPEC, _B_SPEC,
        ],
        out_specs=_ROW_SPEC,
        out_shape=jax.ShapeDtypeStruct((NP, D), jnp.float32),
        name="sage_dense2",
    )(S, S1d, h, W_l, W_r, b)


@functools.cache
def _seg_sum(width):
    return jax.jit(_make_seg_sum(width))


def kernel(x, edge_index, W1_l, W1_r, b1, W2_l, W2_r, b2):
    src = edge_index[0].astype(jnp.int32)
    dst = edge_index[1].astype(jnp.int32)
    ones = jnp.ones((N, 1), jnp.bfloat16)
    x16 = x.astype(jnp.bfloat16)
    x_aug = jnp.concatenate(
        [x16, ones, jnp.zeros((N, DA - D - 1), jnp.bfloat16)], axis=1)
    x_aug = jnp.pad(x_aug, ((0, NP - N), (0, 0)))
    x_pad = jnp.pad(x, ((0, NP - N), (0, 0)))
    (S1,) = _seg_sum(DA)(x_aug, src, dst)
    h, h16 = _layer1(S1, x_pad, W1_l, W1_r, b1.reshape(1, D))
    (S2,) = _seg_sum(D)(h16, src, dst)
    out = _layer2(S2, S1[:, :, D:], h, W2_l, W2_r, b2.reshape(1, D))
    return out[:N]
